# baseline (speedup 1.0000x reference)
import functools

import jax
import jax.numpy as jnp
import numpy as np
from jax import lax
from jax.experimental import pallas as pl
from jax.experimental.pallas import tpu as pltpu
from jax.experimental.pallas import tpu_sc as plsc

_HALF_PI = float(np.pi / 2.0)
_NC = 2
_NS = 16
_NW = _NC * _NS
_L = 16
_CHUNK = 256
_PAIRS = [(2 * m, 2 * m + 1) for m in range(6)] + [
    (13 + 2 * m, 14 + 2 * m) for m in range(6)
]


def _sc_group(zbuf, evbuf, g):
    rows = g * _L + lax.iota(jnp.int32, _L)

    def col(c):
        return plsc.load_gather(zbuf, [rows, jnp.full((_L,), c, jnp.int32)])

    x = {c: col(c) for c in range(25) if c != 12}
    ivec = col(12)
    half = jnp.where(ivec >= 0.0, 0.5, -0.5)
    k = (ivec + half).astype(jnp.int32)
    th = (ivec - k.astype(jnp.float32)) * _HALF_PI
    t2 = th * th
    sr = th * (1.0 + t2 * (-1.0 / 6.0 + t2 * (1.0 / 120.0 + t2 * (-1.0 / 5040.0))))
    cr = 1.0 + t2 * (-0.5 + t2 * (1.0 / 24.0 + t2 * (-1.0 / 720.0)))
    q = k & 3
    c0 = jnp.where(q == 0, cr, jnp.where(q == 1, -sr, jnp.where(q == 2, -cr, sr)))
    s0 = jnp.where(q == 0, sr, jnp.where(q == 1, cr, jnp.where(q == 2, -sr, -cr)))
    q0 = x[0]
    q1 = x[1]
    h0 = jnp.abs(c0 * q0 - s0 * q1)
    h1 = jnp.abs(s0 * q0 + c0 * q1)
    h2 = jnp.abs(x[2])
    h3 = jnp.abs(x[3] + x[4] + x[5])
    j = jnp.zeros((_L,), jnp.int32)
    m = h0
    j = jnp.where(h1 > m, 1, j)
    m = jnp.maximum(m, h1)
    j = jnp.where(h2 > m, 2, j)
    m = jnp.maximum(m, h2)
    j = jnp.where(h3 > m, 3, j)
    c = jnp.where(j == 0, c0, jnp.where(j == 1, s0, jnp.where(j == 2, -c0, -s0)))
    s = jnp.where(j == 0, -s0, jnp.where(j == 1, c0, jnp.where(j == 2, s0, -c0)))

    evv = evbuf[pl.ds(g * _L, _L)] != 0

    def put(cidx, vals):
        plsc.store_scatter(
            zbuf, [rows, jnp.full((_L,), cidx, jnp.int32)], vals, mask=evv
        )

    put(12, j.astype(jnp.float32))
    for a, b in _PAIRS:
        put(a, c * x[a] - s * x[b])
        put(b, s * x[a] + c * x[b])


def _sc_body(z_hbm, ev_hbm, out_hbm, zb0, zb1, eb0, eb1, sin0, sin1, sout0, sout1):
    rows_per_worker = z_hbm.shape[0] // _NW
    nchunks = rows_per_worker // _CHUNK
    wid = lax.axis_index("s") * _NC + lax.axis_index("c")
    base = wid * rows_per_worker

    def start_in(kc, zb, eb, sem):
        row0 = base + kc * _CHUNK
        pltpu.make_async_copy(z_hbm.at[pl.ds(row0, _CHUNK)], zb, sem).start()
        pltpu.make_async_copy(ev_hbm.at[pl.ds(row0, _CHUNK)], eb, sem).start()

    def wait_in(zb, eb, sem):
        pltpu.make_async_copy(z_hbm.at[pl.ds(0, _CHUNK)], zb, sem).wait()
        pltpu.make_async_copy(ev_hbm.at[pl.ds(0, _CHUNK)], eb, sem).wait()

    def start_out(kc, zb, sem):
        row0 = base + kc * _CHUNK
        pltpu.make_async_copy(zb, out_hbm.at[pl.ds(row0, _CHUNK)], sem).start()

    def wait_out(zb, sem):
        pltpu.make_async_copy(zb, out_hbm.at[pl.ds(0, _CHUNK)], sem).wait()

    def compute(zb, eb):
        @plsc.parallel_loop(0, _CHUNK // _L, unroll=8)
        def group_body(g):
            _sc_group(zb, eb, g)

    start_in(0, zb0, eb0, sin0)

    def chunk_body(kc2, carry):
        k0 = kc2 * 2

        @pl.when(kc2 > 0)
        def _():
            wait_out(zb1, sout1)

        start_in(k0 + 1, zb1, eb1, sin1)
        wait_in(zb0, eb0, sin0)
        compute(zb0, eb0)
        start_out(k0, zb0, sout0)
        wait_in(zb1, eb1, sin1)
        compute(zb1, eb1)
        wait_out(zb0, sout0)

        @pl.when(kc2 + 1 < nchunks // 2)
        def _():
            start_in(k0 + 2, zb0, eb0, sin0)

        start_out(k0 + 1, zb1, sout1)
        return carry

    lax.fori_loop(0, nchunks // 2, chunk_body, 0)
    wait_out(zb1, sout1)


def kernel(t, z, ev):
    del t
    B, D = z.shape
    ev32 = ev.astype(jnp.int32)
    run = functools.partial(
        pl.kernel,
        out_type=jax.ShapeDtypeStruct((B, D), jnp.float32),
        mesh=plsc.VectorSubcoreMesh(core_axis_name="c", subcore_axis_name="s"),
        scratch_types=[
            pltpu.VMEM((_CHUNK, D), jnp.float32),
            pltpu.VMEM((_CHUNK, D), jnp.float32),
            pltpu.VMEM((_CHUNK,), jnp.int32),
            pltpu.VMEM((_CHUNK,), jnp.int32),
            pltpu.SemaphoreType.DMA,
            pltpu.SemaphoreType.DMA,
            pltpu.SemaphoreType.DMA,
            pltpu.SemaphoreType.DMA,
        ],
        compiler_params=pltpu.CompilerParams(
            needs_layout_passes=False, use_tc_tiling_on_sc=True
        ),
    )(_sc_body)
    return run(z, ev32)


# speedup vs baseline: 1.0067x; 1.0067x over previous
import functools

import jax
import jax.numpy as jnp
import numpy as np
from jax import lax
from jax.experimental import pallas as pl
from jax.experimental.pallas import tpu as pltpu
from jax.experimental.pallas import tpu_sc as plsc

_HALF_PI = float(np.pi / 2.0)
_NC = 2
_NS = 16
_NW = _NC * _NS
_L = 16
_CHUNK = 256
_PAIRS = [(2 * m, 2 * m + 1) for m in range(6)] + [
    (13 + 2 * m, 14 + 2 * m) for m in range(6)
]


def _sc_group(zbuf, evbuf, g):
    rows = g * _L + lax.iota(jnp.int32, _L)

    def col(c):
        return plsc.load_gather(zbuf, [rows, jnp.full((_L,), c, jnp.int32)])

    x = {c: col(c) for c in range(25) if c != 12}
    ivec = col(12)
    half = jnp.where(ivec >= 0.0, 0.5, -0.5)
    k = (ivec + half).astype(jnp.int32)
    th = (ivec - k.astype(jnp.float32)) * _HALF_PI
    t2 = th * th
    sr = th * (1.0 + t2 * (-1.0 / 6.0 + t2 * (1.0 / 120.0 + t2 * (-1.0 / 5040.0))))
    cr = 1.0 + t2 * (-0.5 + t2 * (1.0 / 24.0 + t2 * (-1.0 / 720.0)))
    q = k & 3
    c0 = jnp.where(q == 0, cr, jnp.where(q == 1, -sr, jnp.where(q == 2, -cr, sr)))
    s0 = jnp.where(q == 0, sr, jnp.where(q == 1, cr, jnp.where(q == 2, -sr, -cr)))
    q0 = x[0]
    q1 = x[1]
    h0 = jnp.abs(c0 * q0 - s0 * q1)
    h1 = jnp.abs(s0 * q0 + c0 * q1)
    h2 = jnp.abs(x[2])
    h3 = jnp.abs(x[3] + x[4] + x[5])
    j = jnp.zeros((_L,), jnp.int32)
    m = h0
    j = jnp.where(h1 > m, 1, j)
    m = jnp.maximum(m, h1)
    j = jnp.where(h2 > m, 2, j)
    m = jnp.maximum(m, h2)
    j = jnp.where(h3 > m, 3, j)
    c = jnp.where(j == 0, c0, jnp.where(j == 1, s0, jnp.where(j == 2, -c0, -s0)))
    s = jnp.where(j == 0, -s0, jnp.where(j == 1, c0, jnp.where(j == 2, s0, -c0)))

    evv = evbuf[pl.ds(g * _L, _L)] != 0

    def put(cidx, vals):
        plsc.store_scatter(
            zbuf, [rows, jnp.full((_L,), cidx, jnp.int32)], vals, mask=evv
        )

    put(12, j.astype(jnp.float32))
    for a, b in _PAIRS:
        put(a, c * x[a] - s * x[b])
        put(b, s * x[a] + c * x[b])


def _sc_body(z_hbm, ev_hbm, out_hbm, zb0, zb1, eb0, eb1, sin0, sin1, sout0, sout1):
    rows_per_worker = z_hbm.shape[0] // _NW
    nchunks = rows_per_worker // _CHUNK
    wid = lax.axis_index("s") * _NC + lax.axis_index("c")
    base = wid * rows_per_worker

    def start_in(kc, zb, eb, sem):
        row0 = base + kc * _CHUNK
        pltpu.make_async_copy(z_hbm.at[pl.ds(row0, _CHUNK)], zb, sem).start()
        pltpu.make_async_copy(ev_hbm.at[pl.ds(row0, _CHUNK)], eb, sem).start()

    def wait_in(zb, eb, sem):
        pltpu.make_async_copy(z_hbm.at[pl.ds(0, _CHUNK)], zb, sem).wait()
        pltpu.make_async_copy(ev_hbm.at[pl.ds(0, _CHUNK)], eb, sem).wait()

    def start_out(kc, zb, sem):
        row0 = base + kc * _CHUNK
        pltpu.make_async_copy(zb, out_hbm.at[pl.ds(row0, _CHUNK)], sem).start()

    def wait_out(zb, sem):
        pltpu.make_async_copy(zb, out_hbm.at[pl.ds(0, _CHUNK)], sem).wait()

    def compute(zb, eb):
        @plsc.parallel_loop(0, _CHUNK // _L, unroll=4)
        def group_body(g):
            _sc_group(zb, eb, g)

    start_in(0, zb0, eb0, sin0)

    def chunk_body(kc2, carry):
        k0 = kc2 * 2

        @pl.when(kc2 > 0)
        def _():
            wait_out(zb1, sout1)

        start_in(k0 + 1, zb1, eb1, sin1)
        wait_in(zb0, eb0, sin0)
        compute(zb0, eb0)
        start_out(k0, zb0, sout0)
        wait_in(zb1, eb1, sin1)
        compute(zb1, eb1)
        wait_out(zb0, sout0)

        @pl.when(kc2 + 1 < nchunks // 2)
        def _():
            start_in(k0 + 2, zb0, eb0, sin0)

        start_out(k0 + 1, zb1, sout1)
        return carry

    lax.fori_loop(0, nchunks // 2, chunk_body, 0)
    wait_out(zb1, sout1)


def kernel(t, z, ev):
    del t
    B, D = z.shape
    ev32 = ev.astype(jnp.int32)
    run = functools.partial(
        pl.kernel,
        out_type=jax.ShapeDtypeStruct((B, D), jnp.float32),
        mesh=plsc.VectorSubcoreMesh(core_axis_name="c", subcore_axis_name="s"),
        scratch_types=[
            pltpu.VMEM((_CHUNK, D), jnp.float32),
            pltpu.VMEM((_CHUNK, D), jnp.float32),
            pltpu.VMEM((_CHUNK,), jnp.int32),
            pltpu.VMEM((_CHUNK,), jnp.int32),
            pltpu.SemaphoreType.DMA,
            pltpu.SemaphoreType.DMA,
            pltpu.SemaphoreType.DMA,
            pltpu.SemaphoreType.DMA,
        ],
        compiler_params=pltpu.CompilerParams(
            needs_layout_passes=False, use_tc_tiling_on_sc=True
        ),
    )(_sc_body)
    return run(z, ev32)
